# baseline (speedup 1.0000x reference)
import functools

import jax
import jax.numpy as jnp
from jax import lax
from jax.experimental import pallas as pl
from jax.experimental.pallas import tpu as pltpu
from jax.experimental.pallas import tpu_sc as plsc

VOCAB = 1000
EMBED_DIM = 64
D_PAD = 128


def _make_sc_gather():
    info = plsc.get_sparse_core_info()
    nc, ns = info.num_cores, info.num_subcores
    nw = nc * ns
    b_per_w = 32

    mesh = plsc.VectorSubcoreMesh(core_axis_name="c", subcore_axis_name="s")

    @functools.partial(
        pl.kernel,
        mesh=mesh,
        out_type=jax.ShapeDtypeStruct((VOCAB, D_PAD), jnp.float32),
        scratch_types=[
            pltpu.VMEM((b_per_w,), jnp.int32),
            pltpu.VMEM((b_per_w, D_PAD), jnp.float32),
            pltpu.SemaphoreType.DMA,
        ],
    )
    def gather_kernel(table_hbm, idx_hbm, out_hbm, idx_v, rows_v, sem):
        wid = lax.axis_index("s") * nc + lax.axis_index("c")
        base = jnp.minimum(wid * b_per_w, VOCAB - b_per_w)
        pltpu.sync_copy(idx_hbm.at[pl.ds(base, b_per_w)], idx_v)
        pltpu.async_copy(table_hbm.at[idx_v], rows_v, sem).wait()
        pltpu.sync_copy(rows_v, out_hbm.at[pl.ds(base, b_per_w)])

    return gather_kernel


def _gemv_body(br, nblocks, e_hbm, w_ref, b_ref, out_ref, e_vmem, sem):
    i = pl.program_id(0)
    K = VOCAB * EMBED_DIM

    @pl.when(i == 0)
    def _():
        copy = pltpu.make_async_copy(e_hbm, e_vmem, sem)
        copy.start()
        copy.wait()
    w3 = w_ref[...].reshape(br // 8, 8, K)
    e3 = e_vmem[...].reshape(1, 1, K)
    out_ref[i] = jnp.sum(w3 * e3, axis=2)

    @pl.when(i == nblocks - 1)
    def _():
        logits = out_ref[...] + b_ref[...]
        m = jnp.max(logits)
        shifted = logits - m
        lse = jnp.log(jnp.sum(jnp.exp(shifted)))
        out_ref[...] = shifted - lse


def _gemv(e_flat, W, b3, br):
    K = VOCAB * EMBED_DIM
    nblocks = VOCAB // br
    return pl.pallas_call(
        functools.partial(_gemv_body, br, nblocks),
        grid=(nblocks,),
        in_specs=[
            pl.BlockSpec(memory_space=pl.ANY),
            pl.BlockSpec((br, K), lambda i: (i, 0)),
            pl.BlockSpec((nblocks, br // 8, 8), lambda i: (0, 0, 0)),
        ],
        out_specs=pl.BlockSpec((nblocks, br // 8, 8), lambda i: (0, 0, 0)),
        out_shape=jax.ShapeDtypeStruct((nblocks, br // 8, 8), jnp.float32),
        scratch_shapes=[
            pltpu.VMEM((1, K), jnp.float32),
            pltpu.SemaphoreType.DMA,
        ],
    )(e_flat, W, b3)


def kernel(x, embed, W, b):
    x = x.astype(jnp.int32)
    embed_pad = jnp.pad(embed, ((0, 0), (0, D_PAD - EMBED_DIM)))

    gather = _make_sc_gather()
    rows = gather(embed_pad, x)
    e_flat = rows[:, :EMBED_DIM].reshape(1, VOCAB * EMBED_DIM)

    br = 8
    nblocks = VOCAB // br
    out = _gemv(e_flat, W, b.reshape(nblocks, br // 8, 8), br=br)
    return out.reshape(1, VOCAB)


# speedup vs baseline: 1.6312x; 1.6312x over previous
import functools

import jax
import jax.numpy as jnp
from jax import lax
from jax.experimental import pallas as pl
from jax.experimental.pallas import tpu as pltpu
from jax.experimental.pallas import tpu_sc as plsc

VOCAB = 1000
EMBED_DIM = 64
D_PAD = 128


def _make_sc_gather():
    info = plsc.get_sparse_core_info()
    nc, ns = info.num_cores, info.num_subcores
    nw = nc * ns
    b_per_w = 32

    mesh = plsc.VectorSubcoreMesh(core_axis_name="c", subcore_axis_name="s")

    @functools.partial(
        pl.kernel,
        mesh=mesh,
        out_type=jax.ShapeDtypeStruct((VOCAB, D_PAD), jnp.float32),
        scratch_types=[
            pltpu.VMEM((b_per_w,), jnp.int32),
            pltpu.VMEM((b_per_w, D_PAD), jnp.float32),
            pltpu.SemaphoreType.DMA,
        ],
    )
    def gather_kernel(table_hbm, idx_hbm, out_hbm, idx_v, rows_v, sem):
        wid = lax.axis_index("s") * nc + lax.axis_index("c")
        base = jnp.minimum(wid * b_per_w, VOCAB - b_per_w)
        pltpu.sync_copy(idx_hbm.at[pl.ds(base, b_per_w)], idx_v)
        pltpu.async_copy(table_hbm.at[idx_v], rows_v, sem).wait()
        pltpu.sync_copy(rows_v, out_hbm.at[pl.ds(base, b_per_w)])

    return gather_kernel


def _gemv_body(br, nblocks, e_hbm, w_ref, b_ref, out_ref, e_vmem, sem):
    i = pl.program_id(0)
    K = VOCAB * EMBED_DIM

    @pl.when(i == 0)
    def _():
        copy = pltpu.make_async_copy(e_hbm, e_vmem, sem)
        copy.start()
        copy.wait()
    w3 = w_ref[...].reshape(br // 8, 8, K)
    e3 = e_vmem[...].reshape(1, 1, K)
    out_ref[i] = jnp.sum(w3 * e3, axis=2)

    @pl.when(i == nblocks - 1)
    def _():
        logits = out_ref[...] + b_ref[...]
        m = jnp.max(logits)
        shifted = logits - m
        lse = jnp.log(jnp.sum(jnp.exp(shifted)))
        out_ref[...] = shifted - lse


def _gemv(e_flat, W, b3, br):
    K = VOCAB * EMBED_DIM
    nblocks = VOCAB // br
    return pl.pallas_call(
        functools.partial(_gemv_body, br, nblocks),
        grid=(nblocks,),
        in_specs=[
            pl.BlockSpec(memory_space=pl.ANY),
            pl.BlockSpec((br, K), lambda i: (i, 0)),
            pl.BlockSpec((nblocks, br // 8, 8), lambda i: (0, 0, 0)),
        ],
        out_specs=pl.BlockSpec((nblocks, br // 8, 8), lambda i: (0, 0, 0)),
        out_shape=jax.ShapeDtypeStruct((nblocks, br // 8, 8), jnp.float32),
        scratch_shapes=[
            pltpu.VMEM((1, K), jnp.float32),
            pltpu.SemaphoreType.DMA,
        ],
    )(e_flat, W, b3)


def kernel(x, embed, W, b):
    x = x.astype(jnp.int32)
    embed_pad = jnp.pad(embed, ((0, 0), (0, D_PAD - EMBED_DIM)))

    gather = _make_sc_gather()
    rows = gather(embed_pad, x)
    e_flat = rows[:, :EMBED_DIM].reshape(1, VOCAB * EMBED_DIM)

    br = 40
    nblocks = VOCAB // br
    out = _gemv(e_flat, W, b.reshape(nblocks, br // 8, 8), br=br)
    return out.reshape(1, VOCAB)


# speedup vs baseline: 1.6525x; 1.0130x over previous
import functools

import jax
import jax.numpy as jnp
from jax import lax
from jax.experimental import pallas as pl
from jax.experimental.pallas import tpu as pltpu
from jax.experimental.pallas import tpu_sc as plsc

VOCAB = 1000
EMBED_DIM = 64
D_PAD = 128


def _make_sc_gather():
    info = plsc.get_sparse_core_info()
    nc, ns = info.num_cores, info.num_subcores
    nc = 1
    nw = nc * ns
    b_per_w = 64

    mesh = plsc.VectorSubcoreMesh(
        core_axis_name="c", subcore_axis_name="s", num_cores=nc
    )

    @functools.partial(
        pl.kernel,
        mesh=mesh,
        out_type=jax.ShapeDtypeStruct((VOCAB, D_PAD), jnp.float32),
        scratch_types=[
            pltpu.VMEM((b_per_w,), jnp.int32),
            pltpu.VMEM((b_per_w, D_PAD), jnp.float32),
            pltpu.SemaphoreType.DMA,
        ],
    )
    def gather_kernel(table_hbm, idx_hbm, out_hbm, idx_v, rows_v, sem):
        wid = lax.axis_index("s") * nc + lax.axis_index("c")
        base = jnp.minimum(wid * b_per_w, VOCAB - b_per_w)
        pltpu.sync_copy(idx_hbm.at[pl.ds(base, b_per_w)], idx_v)
        pltpu.async_copy(table_hbm.at[idx_v], rows_v, sem).wait()
        pltpu.sync_copy(rows_v, out_hbm.at[pl.ds(base, b_per_w)])

    return gather_kernel


def _gemv_body(br, nblocks, e_hbm, w_ref, b_ref, out_ref, e_vmem, sem):
    i = pl.program_id(0)
    K = VOCAB * EMBED_DIM

    @pl.when(i == 0)
    def _():
        copy = pltpu.make_async_copy(e_hbm, e_vmem, sem)
        copy.start()
        copy.wait()
    w3 = w_ref[...].reshape(br // 8, 8, K)
    e3 = e_vmem[...].reshape(1, 1, K)
    out_ref[i] = jnp.sum(w3 * e3, axis=2)

    @pl.when(i == nblocks - 1)
    def _():
        logits = out_ref[...] + b_ref[...]
        m = jnp.max(logits)
        shifted = logits - m
        lse = jnp.log(jnp.sum(jnp.exp(shifted)))
        out_ref[...] = shifted - lse


def _gemv(e_flat, W, b3, br):
    K = VOCAB * EMBED_DIM
    nblocks = VOCAB // br
    return pl.pallas_call(
        functools.partial(_gemv_body, br, nblocks),
        grid=(nblocks,),
        in_specs=[
            pl.BlockSpec(memory_space=pl.ANY),
            pl.BlockSpec((br, K), lambda i: (i, 0)),
            pl.BlockSpec((nblocks, br // 8, 8), lambda i: (0, 0, 0)),
        ],
        out_specs=pl.BlockSpec((nblocks, br // 8, 8), lambda i: (0, 0, 0)),
        out_shape=jax.ShapeDtypeStruct((nblocks, br // 8, 8), jnp.float32),
        scratch_shapes=[
            pltpu.VMEM((1, K), jnp.float32),
            pltpu.SemaphoreType.DMA,
        ],
    )(e_flat, W, b3)


def kernel(x, embed, W, b):
    x = x.astype(jnp.int32)
    embed_pad = jnp.pad(embed, ((0, 0), (0, D_PAD - EMBED_DIM)))

    gather = _make_sc_gather()
    rows = gather(embed_pad, x)
    e_flat = rows[:, :EMBED_DIM].reshape(1, VOCAB * EMBED_DIM)

    br = 40
    nblocks = VOCAB // br
    out = _gemv(e_flat, W, b.reshape(nblocks, br // 8, 8), br=br)
    return out.reshape(1, VOCAB)
